# baseline (speedup 1.0000x reference)
import jax
import jax.numpy as jnp
from jax.experimental import pallas as pl
from jax.experimental.pallas import tpu as pltpu

B, L, D, DO = 16, 2048, 1024, 128
TL = 1024
NB = L // TL


def _body(n_ref, x_ref, W1_ref, b1_ref, W2_ref, b2_ref, out_ref, acc_ref):
    b = pl.program_id(0)
    j = pl.program_id(1)
    n = n_ref[b]

    @pl.when(j == 0)
    def _():
        acc_ref[...] = jnp.zeros_like(acc_ref)

    @pl.when(j * TL < n)
    def _():
        xb = x_ref[0]
        y = jnp.dot(xb, W1_ref[...], preferred_element_type=jnp.float32)
        y = jnp.maximum(y + b1_ref[...], 0.0)
        rows = j * TL + jax.lax.broadcasted_iota(jnp.int32, (TL, 1), 0)
        y = jnp.where(rows < n, y, 0.0)
        acc_ref[...] += jnp.sum(y, axis=0, keepdims=True)

    @pl.when(j == NB - 1)
    def _():
        pooled = acc_ref[...] / n.astype(jnp.float32)
        out_ref[pl.ds(b, 1), :] = (
            jnp.dot(pooled, W2_ref[...], preferred_element_type=jnp.float32)
            + b2_ref[...]
        )


def kernel(x, n_instances, W1, b1, W2, b2):
    n = n_instances.astype(jnp.int32)
    b1r = b1.reshape(1, D)
    b2r = b2.reshape(1, DO)

    grid_spec = pltpu.PrefetchScalarGridSpec(
        num_scalar_prefetch=1,
        grid=(B, NB),
        in_specs=[
            pl.BlockSpec(
                (1, TL, D),
                lambda b, j, n_ref: (b, jnp.minimum(j, (n_ref[b] - 1) // TL), 0),
            ),
            pl.BlockSpec((D, D), lambda b, j, n_ref: (0, 0)),
            pl.BlockSpec((1, D), lambda b, j, n_ref: (0, 0)),
            pl.BlockSpec((D, DO), lambda b, j, n_ref: (0, 0)),
            pl.BlockSpec((1, DO), lambda b, j, n_ref: (0, 0)),
        ],
        out_specs=pl.BlockSpec((B, DO), lambda b, j, n_ref: (0, 0)),
        scratch_shapes=[pltpu.VMEM((1, D), jnp.float32)],
    )

    return pl.pallas_call(
        _body,
        grid_spec=grid_spec,
        out_shape=jax.ShapeDtypeStruct((B, DO), jnp.float32),
        compiler_params=pltpu.CompilerParams(
            dimension_semantics=("parallel", "arbitrary"),
        ),
    )(n, x, W1, b1r, W2, b2r)


# speedup vs baseline: 1.0178x; 1.0178x over previous
import jax
import jax.numpy as jnp
from jax.experimental import pallas as pl
from jax.experimental.pallas import tpu as pltpu

B, L, D, DO = 16, 2048, 1024, 128
TL = 512
NB = L // TL


def _body(n_ref, x_ref, W1_ref, b1_ref, W2_ref, b2_ref, out_ref, acc_ref):
    b = pl.program_id(0)
    j = pl.program_id(1)
    n = n_ref[b]

    @pl.when(j == 0)
    def _():
        acc_ref[...] = jnp.zeros_like(acc_ref)

    @pl.when(j * TL < n)
    def _():
        xb = x_ref[0].astype(jnp.bfloat16)
        y = jnp.dot(xb, W1_ref[...], preferred_element_type=jnp.float32)
        y = jnp.maximum(y + b1_ref[...], 0.0)
        rows = j * TL + jax.lax.broadcasted_iota(jnp.int32, (TL, 1), 0)
        y = jnp.where(rows < n, y, 0.0)
        acc_ref[...] += jnp.sum(y, axis=0, keepdims=True)

    @pl.when(j == NB - 1)
    def _():
        pooled = acc_ref[...] / n.astype(jnp.float32)
        out_ref[pl.ds(b, 1), :] = (
            jnp.dot(pooled, W2_ref[...], preferred_element_type=jnp.float32)
            + b2_ref[...]
        )


def kernel(x, n_instances, W1, b1, W2, b2):
    n = n_instances.astype(jnp.int32)
    W1c = W1.astype(jnp.bfloat16)
    b1r = b1.reshape(1, D)
    b2r = b2.reshape(1, DO)

    grid_spec = pltpu.PrefetchScalarGridSpec(
        num_scalar_prefetch=1,
        grid=(B, NB),
        in_specs=[
            pl.BlockSpec(
                (1, TL, D),
                lambda b, j, n_ref: (b, jnp.minimum(j, (n_ref[b] - 1) // TL), 0),
            ),
            pl.BlockSpec((D, D), lambda b, j, n_ref: (0, 0)),
            pl.BlockSpec((1, D), lambda b, j, n_ref: (0, 0)),
            pl.BlockSpec((D, DO), lambda b, j, n_ref: (0, 0)),
            pl.BlockSpec((1, DO), lambda b, j, n_ref: (0, 0)),
        ],
        out_specs=pl.BlockSpec((B, DO), lambda b, j, n_ref: (0, 0)),
        scratch_shapes=[pltpu.VMEM((1, D), jnp.float32)],
    )

    return pl.pallas_call(
        _body,
        grid_spec=grid_spec,
        out_shape=jax.ShapeDtypeStruct((B, DO), jnp.float32),
        compiler_params=pltpu.CompilerParams(
            dimension_semantics=("parallel", "arbitrary"),
        ),
    )(n, x, W1c, b1r, W2, b2r)


# speedup vs baseline: 1.0575x; 1.0390x over previous
import jax
import jax.numpy as jnp
from jax.experimental import pallas as pl
from jax.experimental.pallas import tpu as pltpu

B, L, D, DO = 16, 2048, 1024, 128
TL = 512
NB = L // TL


def _body(n_ref, x_ref, W1_ref, b1_ref, W2_ref, b2_ref, out_ref, acc_ref):
    b = pl.program_id(0)
    j = pl.program_id(1)
    n = n_ref[b]

    @pl.when(j == 0)
    def _():
        acc_ref[...] = jnp.zeros_like(acc_ref)

    @pl.when(j * TL < n)
    def _():
        xb = x_ref[0]
        y = jnp.dot(xb, W1_ref[...], preferred_element_type=jnp.float32)
        y = jnp.maximum(y + b1_ref[...], 0.0)
        rows = j * TL + jax.lax.broadcasted_iota(jnp.int32, (TL, 1), 0)
        y = jnp.where(rows < n, y, 0.0)
        acc_ref[...] += jnp.sum(y, axis=0, keepdims=True)

    @pl.when(j == NB - 1)
    def _():
        pooled = acc_ref[...] / n.astype(jnp.float32)
        out_ref[pl.ds(b, 1), :] = (
            jnp.dot(pooled, W2_ref[...], preferred_element_type=jnp.float32)
            + b2_ref[...]
        )


def kernel(x, n_instances, W1, b1, W2, b2):
    n = n_instances.astype(jnp.int32)
    W1c = W1
    b1r = b1.reshape(1, D)
    b2r = b2.reshape(1, DO)

    grid_spec = pltpu.PrefetchScalarGridSpec(
        num_scalar_prefetch=1,
        grid=(B, NB),
        in_specs=[
            pl.BlockSpec(
                (1, TL, D),
                lambda b, j, n_ref: (b, jnp.minimum(j, (n_ref[b] - 1) // TL), 0),
            ),
            pl.BlockSpec((D, D), lambda b, j, n_ref: (0, 0)),
            pl.BlockSpec((1, D), lambda b, j, n_ref: (0, 0)),
            pl.BlockSpec((D, DO), lambda b, j, n_ref: (0, 0)),
            pl.BlockSpec((1, DO), lambda b, j, n_ref: (0, 0)),
        ],
        out_specs=pl.BlockSpec((B, DO), lambda b, j, n_ref: (0, 0)),
        scratch_shapes=[pltpu.VMEM((1, D), jnp.float32)],
    )

    return pl.pallas_call(
        _body,
        grid_spec=grid_spec,
        out_shape=jax.ShapeDtypeStruct((B, DO), jnp.float32),
        compiler_params=pltpu.CompilerParams(
            dimension_semantics=("parallel", "arbitrary"),
        ),
    )(n, x, W1c, b1r, W2, b2r)
